# baseline (speedup 1.0000x reference)
import functools

import jax
import jax.numpy as jnp
from jax import lax
from jax.experimental import pallas as pl
from jax.experimental.pallas import tpu as pltpu
from jax.experimental.pallas import tpu_sc as plsc

VOCAB = 8673025
DIM = 4
L = 3335
B = 1024

NC = 2
NS = 16
NW = NC * NS
RPW = B // NW

CHUNK = 128
NCHUNK = 27
LP = NCHUNK * CHUNK
NVEC = LP * DIM // 16


def _body(table_hbm, idx_hbm, wt_hbm, dwb_hbm, out_hbm,
          wt_v, dw_v, idx_v, rows_v, xbuf, outv, sem0, sem1):
  wid = lax.axis_index("s") * NC + lax.axis_index("c")
  base_b = wid * RPW
  sems = (sem0, sem1)

  pltpu.sync_copy(wt_hbm, wt_v)
  pltpu.sync_copy(dwb_hbm, dw_v)

  iota16 = lax.iota(jnp.int32, 16)
  rowpat = iota16 // 4
  colpat = iota16 % 4

  def fire(buf, g):
    pltpu.sync_copy(idx_hbm.at[base_b + g], idx_v.at[buf])
    for j in range(NCHUNK):
      pltpu.async_copy(table_hbm.at[idx_v.at[buf, j]],
                       rows_v.at[buf, pl.ds(j * CHUNK, CHUNK)],
                       sems[buf])

  def drain(buf):
    for j in range(NCHUNK):
      pltpu.make_async_copy(table_hbm.at[idx_v.at[buf, j]],
                            rows_v.at[buf, pl.ds(j * CHUNK, CHUNK)],
                            sems[buf]).wait()

  fire(0, 0)

  bufsplat = (jnp.zeros((16,), jnp.int32), jnp.ones((16,), jnp.int32))

  def group(g, buf):
    @pl.when(g + 1 < RPW)
    def _():
      fire(1 - buf, g + 1)
    drain(buf)

    def fma(v, acc):
      wvec = wt_v[pl.ds(v * 16, 16)]
      ridx = rowpat + v * 4
      gat = plsc.load_gather(rows_v, [bufsplat[buf], ridx, colpat])
      return acc + gat * wvec

    acc = lax.fori_loop(0, NVEC, fma, jnp.zeros((16,), jnp.float32))
    xbuf[pl.ds(g * 16, 16)] = acc

  def outer(t, carry):
    group(2 * t, 0)
    group(2 * t + 1, 1)
    return carry

  lax.fori_loop(0, RPW // 2, outer, jnp.int32(0))

  for h in range(2):
    ys = []
    for c in range(4):
      xc = jnp.zeros((16,), jnp.float32)
      for gi in range(4):
        xc = xc + plsc.load_gather(
            xbuf, [h * 256 + iota16 * 16 + (c + 4 * gi)])
      relu6 = jnp.minimum(jnp.maximum(xc + 3.0, 0.0), 6.0)
      ys.append(xc * relu6 * (1.0 / 6.0))
    z = jnp.zeros((16,), jnp.float32)
    for c in range(4):
      z = z + ys[c] * dw_v[pl.ds(c * 16, 16)]
    e = jnp.exp(-2.0 * jnp.abs(z))
    outv[pl.ds(h * 16, 16)] = jnp.sign(z) * (1.0 - e) / (1.0 + e)

  pltpu.sync_copy(outv, out_hbm.at[pl.ds(base_b, RPW)])


@jax.jit
def _run(table, idx3, wt, dwb):
  call = pl.kernel(
      _body,
      out_type=jax.ShapeDtypeStruct((B,), jnp.float32),
      mesh=plsc.VectorSubcoreMesh(core_axis_name="c", subcore_axis_name="s",
                                  num_cores=NC, num_subcores=NS),
      scratch_types=[
          pltpu.VMEM((LP * DIM,), jnp.float32),
          pltpu.VMEM((64,), jnp.float32),
          pltpu.VMEM((2, NCHUNK, CHUNK), jnp.int32),
          pltpu.VMEM((2, LP, 8), jnp.float32),
          pltpu.VMEM((512,), jnp.float32),
          pltpu.VMEM((32,), jnp.float32),
          pltpu.SemaphoreType.DMA,
          pltpu.SemaphoreType.DMA,
      ],
      compiler_params=pltpu.CompilerParams(use_tc_tiling_on_sc=False,
                                           needs_layout_passes=False),
  )
  return call(table, idx3, wt, dwb)


def kernel(inputs, table, conv_w, dense_w):
  table8 = jnp.pad(table.astype(jnp.float32), ((0, 0), (0, 4)))
  idx = inputs.astype(jnp.int32)
  idx3 = jnp.pad(idx, ((0, 0), (0, LP - L))).reshape(B, NCHUNK, CHUNK)
  wt = jnp.pad(conv_w.astype(jnp.float32), ((0, 0), (0, LP - L)))
  wt = wt.T.reshape(-1)
  dwb = jnp.broadcast_to(dense_w.astype(jnp.float32).reshape(DIM, 1),
                         (DIM, 16)).reshape(-1)
  out = _run(table8, idx3, wt, dwb)
  return out.reshape(B, 1)


# speedup vs baseline: 1.1548x; 1.1548x over previous
import jax
import jax.numpy as jnp
from jax import lax
from jax.experimental import pallas as pl
from jax.experimental.pallas import tpu as pltpu
from jax.experimental.pallas import tpu_sc as plsc

VOCAB = 8673025
DIM = 4
L = 3335
B = 1024

NC = 2
NS = 16
NW = NC * NS
RPW = B // NW

CHUNK = 128
NCHUNK = 27
LP = NCHUNK * CHUNK
NVEC = LP * DIM // 16

NPAIR = (VOCAB - 1) // 2
SPARE = LP * 8
WTX = LP * DIM + 16


def _body(table_hbm, idx_hbm, aux_hbm, wt_hbm, dwb_hbm, out_hbm,
          wt_v, dw_v, idx_v, aux_v, rows_v, xbuf, outv, sem0, sem1):
  wid = lax.axis_index("s") * NC + lax.axis_index("c")
  base_b = wid * RPW
  sems = (sem0, sem1)

  pltpu.sync_copy(wt_hbm, wt_v)
  pltpu.sync_copy(dwb_hbm, dw_v)

  iota16 = lax.iota(jnp.int32, 16)
  rowpat = iota16 // 4
  colpat = iota16 % 4

  lastrow = wt_v[pl.ds(LP * DIM, 16)]
  for buf in range(2):
    plsc.store_scatter(rows_v,
                       [jnp.full((16,), buf, jnp.int32),
                        jnp.full((16,), LP, jnp.int32), iota16],
                       lastrow, mask=iota16 < 4)

  def fire(buf, g):
    pltpu.sync_copy(idx_hbm.at[base_b + g], idx_v.at[buf])
    pltpu.sync_copy(aux_hbm.at[base_b + g], aux_v.at[buf])
    for j in range(NCHUNK):
      pltpu.async_copy(table_hbm.at[idx_v.at[buf, j]],
                       rows_v.at[buf, pl.ds(j * CHUNK, CHUNK)],
                       sems[buf])

  def drain(buf):
    for j in range(NCHUNK):
      pltpu.make_async_copy(table_hbm.at[idx_v.at[buf, j]],
                            rows_v.at[buf, pl.ds(j * CHUNK, CHUNK)],
                            sems[buf]).wait()

  fire(0, 0)

  bufsplat = (jnp.zeros((16,), jnp.int32), jnp.ones((16,), jnp.int32))

  def group(g, buf):
    @pl.when(g + 1 < RPW)
    def _():
      fire(1 - buf, g + 1)
    drain(buf)

    def fma(v, acc):
      wvec = wt_v[pl.ds(v * 16, 16)]
      avec = plsc.load_gather(aux_v.at[buf], [rowpat + v * 4])
      flat = avec + colpat
      gat = plsc.load_gather(rows_v, [bufsplat[buf], flat >> 3, flat & 7])
      return acc + gat * wvec

    acc = lax.fori_loop(0, NVEC, fma, jnp.zeros((16,), jnp.float32))
    xbuf[pl.ds(g * 16, 16)] = acc

  def outer(t, carry):
    group(2 * t, 0)
    group(2 * t + 1, 1)
    return carry

  lax.fori_loop(0, RPW // 2, outer, jnp.int32(0))

  for h in range(2):
    ys = []
    for c in range(4):
      xc = jnp.zeros((16,), jnp.float32)
      for gi in range(4):
        xc = xc + plsc.load_gather(
            xbuf, [h * 256 + iota16 * 16 + (c + 4 * gi)])
      relu6 = jnp.minimum(jnp.maximum(xc + 3.0, 0.0), 6.0)
      ys.append(xc * relu6 * (1.0 / 6.0))
    z = jnp.zeros((16,), jnp.float32)
    for c in range(4):
      z = z + ys[c] * dw_v[pl.ds(c * 16, 16)]
    e = jnp.exp(-2.0 * jnp.abs(z))
    outv[pl.ds(h * 16, 16)] = jnp.sign(z) * (1.0 - e) / (1.0 + e)

  pltpu.sync_copy(outv, out_hbm.at[pl.ds(base_b, RPW)])


@jax.jit
def _run(table2, idx3, aux, wt, dwb):
  call = pl.kernel(
      _body,
      out_type=jax.ShapeDtypeStruct((B,), jnp.float32),
      mesh=plsc.VectorSubcoreMesh(core_axis_name="c", subcore_axis_name="s",
                                  num_cores=NC, num_subcores=NS),
      scratch_types=[
          pltpu.VMEM((WTX,), jnp.float32),
          pltpu.VMEM((64,), jnp.float32),
          pltpu.VMEM((2, NCHUNK, CHUNK), jnp.int32),
          pltpu.VMEM((2, LP), jnp.int32),
          pltpu.VMEM((2, LP + 2, 8), jnp.float32),
          pltpu.VMEM((512,), jnp.float32),
          pltpu.VMEM((32,), jnp.float32),
          pltpu.SemaphoreType.DMA,
          pltpu.SemaphoreType.DMA,
      ],
      compiler_params=pltpu.CompilerParams(use_tc_tiling_on_sc=False,
                                           needs_layout_passes=False),
  )
  return call(table2, idx3, aux, wt, dwb)


def kernel(inputs, table, conv_w, dense_w):
  tablef = table.astype(jnp.float32)
  table2 = tablef[:2 * NPAIR].reshape(NPAIR, 8)
  oidx = inputs.astype(jnp.int32)
  oidx = jnp.pad(oidx, ((0, 0), (0, LP - L)))
  islast = oidx == (VOCAB - 1)
  gidx = jnp.where(islast, 0, oidx >> 1)
  l8 = (jnp.arange(LP, dtype=jnp.int32) * 8)[None, :]
  aux = jnp.where(islast, SPARE, l8 + (oidx & 1) * 4)
  idx3 = gidx.reshape(B, NCHUNK, CHUNK)
  wt = jnp.pad(conv_w.astype(jnp.float32), ((0, 0), (0, LP - L)))
  wt = wt.T.reshape(-1)
  wtx = jnp.concatenate([wt, tablef[-1], jnp.zeros(12, jnp.float32)])
  dwb = jnp.broadcast_to(dense_w.astype(jnp.float32).reshape(DIM, 1),
                         (DIM, 16)).reshape(-1)
  out = _run(table2, idx3, aux, wtx, dwb)
  return out.reshape(B, 1)


# speedup vs baseline: 3.9223x; 3.3965x over previous
import jax
import jax.numpy as jnp
from jax import lax
from jax.experimental import pallas as pl
from jax.experimental.pallas import tpu as pltpu
from jax.experimental.pallas import tpu_sc as plsc

VOCAB = 8673025
DIM = 4
L = 3335
B = 1024

NC = 2
NS = 16
NW = NC * NS
RPW = B // NW

CHUNK = 128
LP = 3456
NCHUNK = LP // CHUNK
EPR = LP * DIM
CVEC = LP // 16
UNROLL = 8


def _body(table_hbm, idx_hbm, wt_hbm, dwb_hbm, out_hbm,
          wt_v, dw_v, idx_v, rows_v, xbuf, outv, sem0, sem1):
  wid = lax.axis_index("s") * NC + lax.axis_index("c")
  base_b = wid * RPW
  sems = (sem0, sem1)

  pltpu.sync_copy(wt_hbm, wt_v)
  pltpu.sync_copy(dwb_hbm, dw_v)

  iota16 = lax.iota(jnp.int32, 16)

  def fire(buf, g):
    pltpu.sync_copy(idx_hbm.at[pl.ds((base_b + g) * LP, LP)], idx_v.at[buf])
    for c in range(DIM):
      for j in range(NCHUNK):
        pltpu.async_copy(
            table_hbm.at[c].at[idx_v.at[buf, pl.ds(j * CHUNK, CHUNK)]],
            rows_v.at[buf, pl.ds(c * LP + j * CHUNK, CHUNK)],
            sems[buf])

  def drain(buf):
    for c in range(DIM):
      for j in range(NCHUNK):
        pltpu.make_async_copy(
            table_hbm.at[c].at[idx_v.at[buf, pl.ds(j * CHUNK, CHUNK)]],
            rows_v.at[buf, pl.ds(c * LP + j * CHUNK, CHUNK)],
            sems[buf]).wait()

  fire(0, 0)

  def group(g, buf):
    @pl.when(g + 1 < RPW)
    def _():
      fire(1 - buf, g + 1)
    drain(buf)

    for c in range(DIM):
      def fma(s, accs):
        accs = list(accs)
        for u in range(UNROLL):
          off = (c * CVEC + s * UNROLL + u) * 16
          accs[u % 4] = accs[u % 4] + (rows_v[buf, pl.ds(off, 16)]
                                       * wt_v[pl.ds(off, 16)])
        return tuple(accs)

      zero = jnp.zeros((16,), jnp.float32)
      accs = lax.fori_loop(0, CVEC // UNROLL, fma, (zero, zero, zero, zero))
      acc = (accs[0] + accs[1]) + (accs[2] + accs[3])
      xbuf[pl.ds((g * 4 + c) * 16, 16)] = acc

  def outer(t, carry):
    group(2 * t, 0)
    group(2 * t + 1, 1)
    return carry

  lax.fori_loop(0, RPW // 2, outer, jnp.int32(0))

  for h in range(2):
    ys = []
    for c in range(4):
      xc = jnp.zeros((16,), jnp.float32)
      for k in range(16):
        xc = xc + plsc.load_gather(
            xbuf, [(h * 16 + iota16) * 64 + c * 16 + k])
      relu6 = jnp.minimum(jnp.maximum(xc + 3.0, 0.0), 6.0)
      ys.append(xc * relu6 * (1.0 / 6.0))
    z = jnp.zeros((16,), jnp.float32)
    for c in range(4):
      z = z + ys[c] * dw_v[pl.ds(c * 16, 16)]
    e = jnp.exp(-2.0 * jnp.abs(z))
    outv[pl.ds(h * 16, 16)] = jnp.sign(z) * (1.0 - e) / (1.0 + e)

  pltpu.sync_copy(outv, out_hbm.at[pl.ds(base_b, RPW)])


@jax.jit
def _run(tT, idxflat, wt, dwb):
  call = pl.kernel(
      _body,
      out_type=jax.ShapeDtypeStruct((B,), jnp.float32),
      mesh=plsc.VectorSubcoreMesh(core_axis_name="c", subcore_axis_name="s",
                                  num_cores=NC, num_subcores=NS),
      scratch_types=[
          pltpu.VMEM((EPR,), jnp.float32),
          pltpu.VMEM((64,), jnp.float32),
          pltpu.VMEM((2, LP), jnp.int32),
          pltpu.VMEM((2, EPR), jnp.float32),
          pltpu.VMEM((RPW * 64,), jnp.float32),
          pltpu.VMEM((32,), jnp.float32),
          pltpu.SemaphoreType.DMA,
          pltpu.SemaphoreType.DMA,
      ],
      compiler_params=pltpu.CompilerParams(use_tc_tiling_on_sc=False,
                                           needs_layout_passes=False),
  )
  return call(tT, idxflat, wt, dwb)


def kernel(inputs, table, conv_w, dense_w):
  tT = table.astype(jnp.float32).T
  oidx = inputs.astype(jnp.int32)
  idxflat = jnp.pad(oidx, ((0, 0), (0, LP - L))).reshape(-1)
  wt = jnp.pad(conv_w.astype(jnp.float32),
               ((0, 0), (0, LP - L))).reshape(-1)
  dwb = jnp.broadcast_to(dense_w.astype(jnp.float32).reshape(DIM, 1),
                         (DIM, 16)).reshape(-1)
  out = _run(tT, idxflat, wt, dwb)
  return out.reshape(B, 1)
